# Initial kernel scaffold; baseline (speedup 1.0000x reference)
import jax
import jax.numpy as jnp
from jax.experimental import pallas as pl


def kernel(inputs):
    raise NotImplementedError("write your pallas kernel here")


# speedup vs baseline: 24.1136x; 24.1136x over previous
import jax
import jax.numpy as jnp
from jax import lax
from jax.experimental import pallas as pl
from jax.experimental.pallas import tpu as pltpu

_B, _H, _W, _C = 4, 512, 512, 32
_PAD = 3


def _focus_body(img_ref, out_ref):
    f32, i32 = jnp.float32, jnp.int32
    H, W = _H, _W
    img = img_ref[0]

    col_max = jnp.max(img, axis=0, keepdims=True)
    row_max = jnp.max(img, axis=1, keepdims=True)
    ax = jnp.round(jax.nn.sigmoid(col_max)).astype(i32)
    ay = jnp.round(jax.nn.sigmoid(row_max)).astype(i32)
    wxv = lax.broadcasted_iota(i32, (1, W), 1)
    wyv = lax.broadcasted_iota(i32, (H, 1), 0)
    m1 = jnp.max(ax * (W - 1 - wxv), axis=1, keepdims=True)
    xm = jnp.where(m1 > 0, W - 1 - m1, 0)
    xM = jnp.max(ax * wxv, axis=1, keepdims=True)
    m3 = jnp.max(ay * (H - 1 - wyv), axis=0, keepdims=True)
    ym = jnp.where(m3 > 0, H - 1 - m3, 0)
    yM = jnp.max(ay * wyv, axis=0, keepdims=True)

    hc = jnp.maximum(yM - ym, 1)
    wc = jnp.maximum(xM - xm, 1)
    hcf = hc.astype(f32)
    wcf = wc.astype(f32)
    zh = jnp.maximum(H - 2 * _PAD, hc).astype(f32)
    zw = jnp.maximum(W - 2 * _PAD, wc).astype(f32)
    scale = jnp.minimum(zh / hcf, zw / wcf)
    rh = jnp.round(scale * hcf).astype(i32)
    rw = jnp.round(scale * wcf).astype(i32)

    oi = lax.broadcasted_iota(i32, (H, 1), 0)
    ri = oi - jnp.maximum(0, (H - rh) // 2) + jnp.maximum(0, (rh - H) // 2)
    valid_r = (ri >= 0) & (ri < rh)
    sy = ri.astype(f32) * (hcf / rh.astype(f32))
    sy = jnp.minimum(sy, hcf - 1.0)
    s_y = jnp.where(valid_r, ym.astype(f32) + sy, -2.0 * H)
    kk = lax.broadcasted_iota(i32, (H, H), 1).astype(f32)
    Ay = jnp.maximum(1.0 - jnp.abs(s_y - kk), 0.0)

    oj = lax.broadcasted_iota(i32, (1, W), 1)
    rj = oj - jnp.maximum(0, (W - rw) // 2) + jnp.maximum(0, (rw - W) // 2)
    valid_c = (rj >= 0) & (rj < rw)
    sx = rj.astype(f32) * (wcf / rw.astype(f32))
    sx = jnp.minimum(sx, wcf - 1.0)
    s_x = jnp.where(valid_c, xm.astype(f32) + sx, -2.0 * W)
    ll = lax.broadcasted_iota(i32, (W, W), 0).astype(f32)
    AxT = jnp.maximum(1.0 - jnp.abs(s_x - ll), 0.0)

    tmp = jnp.dot(Ay, img, preferred_element_type=f32)
    out_ref[0] = jnp.dot(tmp, AxT, preferred_element_type=f32)


def kernel(inputs):
    imgs = jnp.transpose(inputs, (0, 3, 1, 2)).reshape(_B * _C, _H, _W)
    out = pl.pallas_call(
        _focus_body,
        grid=(_B * _C,),
        in_specs=[pl.BlockSpec((1, _H, _W), lambda g: (g, 0, 0))],
        out_specs=pl.BlockSpec((1, _H, _W), lambda g: (g, 0, 0)),
        out_shape=jax.ShapeDtypeStruct((_B * _C, _H, _W), jnp.float32),
        compiler_params=pltpu.CompilerParams(
            dimension_semantics=("arbitrary",),
        ),
        name="focus2d",
    )(imgs)
    return jnp.transpose(out.reshape(_B, _C, _H, _W), (0, 2, 3, 1))
